# baseline (speedup 1.0000x reference)
import jax
import jax.numpy as jnp
from jax.experimental import pallas as pl
from jax.experimental.pallas import tpu as pltpu

N = 100000
D = 128
A = 256
G = 64
B = 4000
NB = N // B


def _fused_kernel(x_ref, b_ref, Wa_ref, ba_ref, Wb_ref, bb_ref, Wc_ref,
                  bc_ref, Wh_ref, bh_ref, out_ref, s_acc, f_acc):
    i = pl.program_id(0)

    @pl.when(i == 0)
    def _init():
        s_acc[...] = jnp.zeros_like(s_acc)
        f_acc[...] = jnp.zeros_like(f_acc)

    x = x_ref[...]
    xb = x.astype(jnp.bfloat16)
    za = jnp.dot(xb, Wa_ref[...],
                 preferred_element_type=jnp.float32)
    a = jnp.tanh(za.astype(jnp.bfloat16) + ba_ref[...])
    zb = jnp.dot(xb, Wb_ref[...],
                 preferred_element_type=jnp.float32)
    tb = jnp.tanh(zb.astype(jnp.bfloat16) + bb_ref[...])
    ab = a + a * tb
    gate = jnp.dot(ab, Wc_ref[...],
                   preferred_element_type=jnp.float32) + bc_ref[...]
    e = jnp.exp(gate)

    bid = b_ref[0].reshape(1, B)
    gid = jax.lax.broadcasted_iota(jnp.int32, (G, B), 0)
    oh = (bid == gid).astype(jnp.float32)
    s_acc[...] += jnp.dot(oh, e, preferred_element_type=jnp.float32)
    f_acc[...] += jnp.dot(oh, x * e, preferred_element_type=jnp.float32)

    @pl.when(i == NB - 1)
    def _heads():
        feat = f_acc[...] / (s_acc[...] + 1e-16)
        out_ref[...] = (jnp.dot(feat, Wh_ref[...],
                                preferred_element_type=jnp.float32)
                        + bh_ref[...])


def kernel(x, batch, Wa, ba, Wb, bb, Wc, bc, Wpk, bpk, Wp, bp):
    batch3 = batch.astype(jnp.int32).reshape(NB, 8, B // 8)
    Wh = jnp.concatenate([Wpk, Wp], axis=1)
    bh = jnp.concatenate([bpk, bp]).reshape(1, 4)

    out = pl.pallas_call(
        _fused_kernel,
        grid=(NB,),
        in_specs=[
            pl.BlockSpec((B, D), lambda i: (i, 0)),
            pl.BlockSpec((1, 8, B // 8), lambda i: (i, 0, 0)),
            pl.BlockSpec((D, A), lambda i: (0, 0)),
            pl.BlockSpec((1, A), lambda i: (0, 0)),
            pl.BlockSpec((D, A), lambda i: (0, 0)),
            pl.BlockSpec((1, A), lambda i: (0, 0)),
            pl.BlockSpec((A, D), lambda i: (0, 0)),
            pl.BlockSpec((1, D), lambda i: (0, 0)),
            pl.BlockSpec((D, 4), lambda i: (0, 0)),
            pl.BlockSpec((1, 4), lambda i: (0, 0)),
        ],
        out_specs=pl.BlockSpec((G, 4), lambda i: (0, 0)),
        out_shape=jax.ShapeDtypeStruct((G, 4), jnp.float32),
        scratch_shapes=[
            pltpu.VMEM((G, D), jnp.float32),
            pltpu.VMEM((G, D), jnp.float32),
        ],
        compiler_params=pltpu.CompilerParams(
            dimension_semantics=("arbitrary",),
        ),
    )(x, batch3,
      Wa.astype(jnp.bfloat16), ba.reshape(1, A).astype(jnp.bfloat16),
      (0.5 * Wb).astype(jnp.bfloat16),
      (0.5 * bb).reshape(1, A).astype(jnp.bfloat16),
      (0.5 * Wc).astype(jnp.bfloat16), bc.reshape(1, D), Wh, bh)
    return out


# speedup vs baseline: 1.1034x; 1.1034x over previous
import jax
import jax.numpy as jnp
from jax.experimental import pallas as pl
from jax.experimental.pallas import tpu as pltpu

N = 100000
D = 128
A = 256
G = 64
B = 5000
NB = N // B


def _fused_kernel(x_ref, b_ref, Wa_ref, ba_ref, Wb_ref, bb_ref, Wc_ref,
                  bc_ref, Wh_ref, bh_ref, out_ref, s_acc, f_acc):
    i = pl.program_id(0)

    @pl.when(i == 0)
    def _init():
        s_acc[...] = jnp.zeros_like(s_acc)
        f_acc[...] = jnp.zeros_like(f_acc)

    x = x_ref[...]
    xb = x.astype(jnp.bfloat16)
    a = jnp.tanh(jnp.dot(xb, Wa_ref[...].astype(jnp.bfloat16),
                         preferred_element_type=jnp.float32) + ba_ref[...])
    zb = jnp.dot(xb, Wb_ref[...].astype(jnp.bfloat16),
                 preferred_element_type=jnp.float32) + bb_ref[...]
    b = 0.5 * (1.0 + jnp.tanh(0.5 * zb))
    gate = jnp.dot((a * b).astype(jnp.bfloat16),
                   Wc_ref[...].astype(jnp.bfloat16),
                   preferred_element_type=jnp.float32) + bc_ref[...]
    e = jnp.exp(gate)

    bid = b_ref[0].reshape(1, B)
    gid = jax.lax.broadcasted_iota(jnp.int32, (G, B), 0)
    oh = (bid == gid).astype(jnp.float32)
    s_acc[...] += jnp.dot(oh, e, preferred_element_type=jnp.float32)
    f_acc[...] += jnp.dot(oh, x * e, preferred_element_type=jnp.float32)

    @pl.when(i == NB - 1)
    def _heads():
        feat = f_acc[...] / (s_acc[...] + 1e-16)
        out_ref[...] = (jnp.dot(feat, Wh_ref[...],
                                preferred_element_type=jnp.float32)
                        + bh_ref[...])


def kernel(x, batch, Wa, ba, Wb, bb, Wc, bc, Wpk, bpk, Wp, bp):
    batch3 = batch.astype(jnp.int32).reshape(NB, 8, B // 8)
    Wh = jnp.concatenate([Wpk, Wp], axis=1)
    bh = jnp.concatenate([bpk, bp]).reshape(1, 4)

    out = pl.pallas_call(
        _fused_kernel,
        grid=(NB,),
        in_specs=[
            pl.BlockSpec((B, D), lambda i: (i, 0)),
            pl.BlockSpec((1, 8, B // 8), lambda i: (i, 0, 0)),
            pl.BlockSpec((D, A), lambda i: (0, 0)),
            pl.BlockSpec((1, A), lambda i: (0, 0)),
            pl.BlockSpec((D, A), lambda i: (0, 0)),
            pl.BlockSpec((1, A), lambda i: (0, 0)),
            pl.BlockSpec((A, D), lambda i: (0, 0)),
            pl.BlockSpec((1, D), lambda i: (0, 0)),
            pl.BlockSpec((D, 4), lambda i: (0, 0)),
            pl.BlockSpec((1, 4), lambda i: (0, 0)),
        ],
        out_specs=pl.BlockSpec((G, 4), lambda i: (0, 0)),
        out_shape=jax.ShapeDtypeStruct((G, 4), jnp.float32),
        scratch_shapes=[
            pltpu.VMEM((G, D), jnp.float32),
            pltpu.VMEM((G, D), jnp.float32),
        ],
        compiler_params=pltpu.CompilerParams(
            dimension_semantics=("arbitrary",),
        ),
    )(x, batch3, Wa, ba.reshape(1, A), Wb, bb.reshape(1, A), Wc,
      bc.reshape(1, D), Wh, bh)
    return out


# speedup vs baseline: 1.1920x; 1.0803x over previous
import jax
import jax.numpy as jnp
from jax.experimental import pallas as pl
from jax.experimental.pallas import tpu as pltpu

N = 100000
D = 128
A = 256
G = 64
B = 10000
NB = N // B


def _fused_kernel(x_ref, b_ref, Wa_ref, ba_ref, Wb_ref, bb_ref, Wc_ref,
                  bc_ref, Wh_ref, bh_ref, out_ref, s_acc, f_acc):
    i = pl.program_id(0)

    @pl.when(i == 0)
    def _init():
        s_acc[...] = jnp.zeros_like(s_acc)
        f_acc[...] = jnp.zeros_like(f_acc)

    x = x_ref[...]
    xb = x.astype(jnp.bfloat16)
    a = jnp.tanh(jnp.dot(xb, Wa_ref[...].astype(jnp.bfloat16),
                         preferred_element_type=jnp.float32) + ba_ref[...])
    zb = jnp.dot(xb, Wb_ref[...].astype(jnp.bfloat16),
                 preferred_element_type=jnp.float32) + bb_ref[...]
    b = 0.5 * (1.0 + jnp.tanh(0.5 * zb))
    gate = jnp.dot((a * b).astype(jnp.bfloat16),
                   Wc_ref[...].astype(jnp.bfloat16),
                   preferred_element_type=jnp.float32) + bc_ref[...]
    e = jnp.exp(gate)

    bid = b_ref[0].reshape(1, B)
    gid = jax.lax.broadcasted_iota(jnp.int32, (G, B), 0)
    oh = (bid == gid).astype(jnp.float32)
    s_acc[...] += jnp.dot(oh, e, preferred_element_type=jnp.float32)
    f_acc[...] += jnp.dot(oh, x * e, preferred_element_type=jnp.float32)

    @pl.when(i == NB - 1)
    def _heads():
        feat = f_acc[...] / (s_acc[...] + 1e-16)
        out_ref[...] = (jnp.dot(feat, Wh_ref[...],
                                preferred_element_type=jnp.float32)
                        + bh_ref[...])


def kernel(x, batch, Wa, ba, Wb, bb, Wc, bc, Wpk, bpk, Wp, bp):
    batch3 = batch.astype(jnp.int32).reshape(NB, 8, B // 8)
    Wh = jnp.concatenate([Wpk, Wp], axis=1)
    bh = jnp.concatenate([bpk, bp]).reshape(1, 4)

    out = pl.pallas_call(
        _fused_kernel,
        grid=(NB,),
        in_specs=[
            pl.BlockSpec((B, D), lambda i: (i, 0)),
            pl.BlockSpec((1, 8, B // 8), lambda i: (i, 0, 0)),
            pl.BlockSpec((D, A), lambda i: (0, 0)),
            pl.BlockSpec((1, A), lambda i: (0, 0)),
            pl.BlockSpec((D, A), lambda i: (0, 0)),
            pl.BlockSpec((1, A), lambda i: (0, 0)),
            pl.BlockSpec((A, D), lambda i: (0, 0)),
            pl.BlockSpec((1, D), lambda i: (0, 0)),
            pl.BlockSpec((D, 4), lambda i: (0, 0)),
            pl.BlockSpec((1, 4), lambda i: (0, 0)),
        ],
        out_specs=pl.BlockSpec((G, 4), lambda i: (0, 0)),
        out_shape=jax.ShapeDtypeStruct((G, 4), jnp.float32),
        scratch_shapes=[
            pltpu.VMEM((G, D), jnp.float32),
            pltpu.VMEM((G, D), jnp.float32),
        ],
        compiler_params=pltpu.CompilerParams(
            dimension_semantics=("arbitrary",),
        ),
    )(x, batch3, Wa, ba.reshape(1, A), Wb, bb.reshape(1, A), Wc,
      bc.reshape(1, D), Wh, bh)
    return out


# speedup vs baseline: 1.2084x; 1.0137x over previous
import jax
import jax.numpy as jnp
from jax.experimental import pallas as pl
from jax.experimental.pallas import tpu as pltpu

N = 100000
D = 128
A = 256
G = 64
B = 20000
NB = N // B


def _fused_kernel(x_ref, b_ref, Wa_ref, ba_ref, Wb_ref, bb_ref, Wc_ref,
                  bc_ref, Wh_ref, bh_ref, out_ref, s_acc, f_acc):
    i = pl.program_id(0)

    @pl.when(i == 0)
    def _init():
        s_acc[...] = jnp.zeros_like(s_acc)
        f_acc[...] = jnp.zeros_like(f_acc)

    x = x_ref[...]
    xb = x.astype(jnp.bfloat16)
    a = jnp.tanh(jnp.dot(xb, Wa_ref[...].astype(jnp.bfloat16),
                         preferred_element_type=jnp.float32) + ba_ref[...])
    zb = jnp.dot(xb, Wb_ref[...].astype(jnp.bfloat16),
                 preferred_element_type=jnp.float32) + bb_ref[...]
    b = 0.5 * (1.0 + jnp.tanh(0.5 * zb))
    gate = jnp.dot((a * b).astype(jnp.bfloat16),
                   Wc_ref[...].astype(jnp.bfloat16),
                   preferred_element_type=jnp.float32) + bc_ref[...]
    e = jnp.exp(gate)

    bid = b_ref[0].reshape(1, B)
    gid = jax.lax.broadcasted_iota(jnp.int32, (G, B), 0)
    oh = (bid == gid).astype(jnp.float32)
    s_acc[...] += jnp.dot(oh, e, preferred_element_type=jnp.float32)
    f_acc[...] += jnp.dot(oh, x * e, preferred_element_type=jnp.float32)

    @pl.when(i == NB - 1)
    def _heads():
        feat = f_acc[...] / (s_acc[...] + 1e-16)
        out_ref[...] = (jnp.dot(feat, Wh_ref[...],
                                preferred_element_type=jnp.float32)
                        + bh_ref[...])


def kernel(x, batch, Wa, ba, Wb, bb, Wc, bc, Wpk, bpk, Wp, bp):
    batch3 = batch.astype(jnp.int32).reshape(NB, 8, B // 8)
    Wh = jnp.concatenate([Wpk, Wp], axis=1)
    bh = jnp.concatenate([bpk, bp]).reshape(1, 4)

    out = pl.pallas_call(
        _fused_kernel,
        grid=(NB,),
        in_specs=[
            pl.BlockSpec((B, D), lambda i: (i, 0)),
            pl.BlockSpec((1, 8, B // 8), lambda i: (i, 0, 0)),
            pl.BlockSpec((D, A), lambda i: (0, 0)),
            pl.BlockSpec((1, A), lambda i: (0, 0)),
            pl.BlockSpec((D, A), lambda i: (0, 0)),
            pl.BlockSpec((1, A), lambda i: (0, 0)),
            pl.BlockSpec((A, D), lambda i: (0, 0)),
            pl.BlockSpec((1, D), lambda i: (0, 0)),
            pl.BlockSpec((D, 4), lambda i: (0, 0)),
            pl.BlockSpec((1, 4), lambda i: (0, 0)),
        ],
        out_specs=pl.BlockSpec((G, 4), lambda i: (0, 0)),
        out_shape=jax.ShapeDtypeStruct((G, 4), jnp.float32),
        scratch_shapes=[
            pltpu.VMEM((G, D), jnp.float32),
            pltpu.VMEM((G, D), jnp.float32),
        ],
        compiler_params=pltpu.CompilerParams(
            dimension_semantics=("arbitrary",),
        ),
    )(x, batch3, Wa, ba.reshape(1, A), Wb, bb.reshape(1, A), Wc,
      bc.reshape(1, D), Wh, bh)
    return out
